# Initial kernel scaffold; baseline (speedup 1.0000x reference)
import jax
import jax.numpy as jnp
from jax.experimental import pallas as pl


def kernel(features, params):
    raise NotImplementedError("write your pallas kernel here")


# speedup vs baseline: 2.6678x; 2.6678x over previous
import functools
import jax
import jax.numpy as jnp
from jax.experimental import pallas as pl
from jax.experimental.pallas import tpu as pltpu

_C = 256
_GROUPS = 32
_GSIZE = _C // _GROUPS
_EPS = 1e-5


def _group_mat():
    r = jax.lax.broadcasted_iota(jnp.int32, (_C, _C), 0) // _GSIZE
    c = jax.lax.broadcasted_iota(jnp.int32, (_C, _C), 1) // _GSIZE
    return (r == c).astype(jnp.float32)


def _tower_kernel(*refs, H, W, n_layers, head_co, bbox):
    if bbox:
        (feat_ref, tw_ref, lp_ref, hw_ref, hb_ref, sc_ref, out_ref,
         pad_ref) = refs
    else:
        feat_ref, tw_ref, lp_ref, hw_ref, hb_ref, out_ref, pad_ref = refs
    N = H * W
    P = _group_mat()

    pad_ref[...] = jnp.zeros_like(pad_ref)
    pad_ref[1:H + 1, 1:W + 1, :] = feat_ref[0]

    def conv9(wref, idx, co):
        acc = None
        for k in range(9):
            di, dj = k // 3, k % 3
            xs = pad_ref[di:di + H, dj:dj + W, :].reshape(N, _C)
            t = jnp.dot(xs, wref[idx + (k,)] if idx else wref[k],
                        preferred_element_type=jnp.float32)
            acc = t if acc is None else acc + t
        return acc

    for layer in range(n_layers):
        acc = conv9(tw_ref, (layer,), _C)
        lp = lp_ref[layer]
        b, gamma, beta = lp[0:1], lp[1:2], lp[2:3]
        gsb, gsb2 = lp[3:4], lp[4:5]
        s = jnp.sum(acc, axis=0, keepdims=True)
        q = jnp.sum(acc * acc, axis=0, keepdims=True)
        stats = jnp.concatenate([s, q, b * s], axis=0)
        gs = jnp.dot(stats, P, preferred_element_type=jnp.float32)
        inv_n = 1.0 / (_GSIZE * N)
        mu = (gs[0:1] + N * gsb) * inv_n
        ey2 = (gs[1:2] + 2.0 * gs[2:3] + N * gsb2) * inv_n
        rstd = jax.lax.rsqrt(ey2 - mu * mu + _EPS)
        sc = rstd * gamma
        sh = (b - mu) * sc + beta
        x = jnp.maximum(acc * sc + sh, 0.0).astype(jnp.bfloat16)
        pad_ref[1:H + 1, 1:W + 1, :] = x.reshape(H, W, _C)

    y = conv9(hw_ref, (), head_co) + hb_ref[0:1]
    if bbox:
        y = jnp.exp(y * sc_ref[...])
    out_ref[0] = y.reshape(H, W, head_co)


def _run_tower(feat, tower_w, lp, head_w, head_b, scale, head_co, bbox):
    B, H, W, _ = feat.shape
    kern = functools.partial(_tower_kernel, H=H, W=W,
                             n_layers=tower_w.shape[0],
                             head_co=head_co, bbox=bbox)
    in_specs = [
        pl.BlockSpec((1, H, W, _C), lambda b: (b, 0, 0, 0)),
        pl.BlockSpec(tower_w.shape, lambda b: (0, 0, 0, 0)),
        pl.BlockSpec(lp.shape, lambda b: (0, 0, 0)),
        pl.BlockSpec(head_w.shape, lambda b: (0, 0, 0)),
        pl.BlockSpec(head_b.shape, lambda b: (0, 0)),
    ]
    args = [feat, tower_w, lp, head_w, head_b]
    if bbox:
        in_specs.append(pl.BlockSpec((1, 1), lambda b: (0, 0)))
        args.append(scale)
    return pl.pallas_call(
        kern,
        grid=(B,),
        in_specs=in_specs,
        out_specs=pl.BlockSpec((1, H, W, head_co), lambda b: (b, 0, 0, 0)),
        out_shape=jax.ShapeDtypeStruct((B, H, W, head_co), jnp.float32),
        scratch_shapes=[pltpu.VMEM((H + 2, W + 2, _C), jnp.bfloat16)],
    )(*args)


def _gs_vec(v):
    return jnp.repeat(v.reshape(_GROUPS, _GSIZE).sum(axis=1), _GSIZE)


def _prep_tower(layers):
    ws, lps = [], []
    for l in layers:
        ws.append(jnp.transpose(l['w'], (2, 3, 1, 0)).reshape(9, _C, _C))
        b, g, beta = l['b'], l['g'], l['beta']
        lps.append(jnp.stack([b, g, beta, _gs_vec(b), _gs_vec(b * b),
                              jnp.zeros_like(b), jnp.zeros_like(b),
                              jnp.zeros_like(b)]))
    return (jnp.stack(ws).astype(jnp.bfloat16),
            jnp.stack(lps).astype(jnp.float32))


def _prep_head(w):
    co = w.shape[0]
    return jnp.transpose(w, (2, 3, 1, 0)).reshape(9, _C, co).astype(
        jnp.bfloat16)


def kernel(features, params):
    cls_tw, cls_lp = _prep_tower(params['cls_tower'])
    box_tw, box_lp = _prep_tower(params['bbox_tower'])
    cls_head_w = _prep_head(jnp.concatenate(
        [params['cls_logits']['w'], params['centerness']['w']], axis=0))
    cls_head_b = jnp.concatenate(
        [params['cls_logits']['b'], params['centerness']['b']])[None, :]
    box_head_w = _prep_head(params['bbox_pred']['w'])
    box_head_b = params['bbox_pred']['b'][None, :]

    logits, bbox, ctr = [], [], []
    for l, f in enumerate(features):
        fx = jnp.transpose(f, (0, 2, 3, 1)).astype(jnp.bfloat16)
        yc = _run_tower(fx, cls_tw, cls_lp, cls_head_w, cls_head_b,
                        None, 81, False)
        sc = params['scales'][l].reshape(1, 1)
        yb = _run_tower(fx, box_tw, box_lp, box_head_w, box_head_b,
                        sc, 4, True)
        logits.append(jnp.transpose(yc[..., :80], (0, 3, 1, 2)))
        ctr.append(jnp.transpose(yc[..., 80:81], (0, 3, 1, 2)))
        bbox.append(jnp.transpose(yb, (0, 3, 1, 2)))
    return tuple(logits), tuple(bbox), tuple(ctr)
